# baseline (speedup 1.0000x reference)
import jax
import jax.numpy as jnp
from jax import lax
from jax.experimental import pallas as pl
from jax.experimental.pallas import tpu as pltpu
from jax.experimental.pallas import tpu_sc as plsc

D = 64
NC, NS = 2, 16
NW = NC * NS
BATCH = 16384
HIST = 50
CHUNK = 256
UB = BATCH // CHUNK
NUNITS = HIST * UB
UPW = NUNITS // NW
NBUF = 2
OBUF = 16384
HSTRIDE = 8 * 128 * 8 * 128


def _emb_body(idx_hbm, tp_hbm, out_hbm, idx_v, pidx_v, par_v, rows_v, obuf,
              sem_g, sem_o):
    wid = lax.axis_index("s") * NC + lax.axis_index("c")
    u0 = wid * UPW
    rowi = lax.iota(jnp.int32, 16)

    def stage_a(u, bx):
        h = u >> 6
        bc2 = u & (UB - 1)
        i0 = h * BATCH + bc2 * CHUNK
        pltpu.sync_copy(idx_hbm.at[pl.ds(i0, CHUNK)], idx_v.at[bx])
        for j in range(CHUNK // 16):
            v = idx_v[bx, pl.ds(j * 16, 16)]
            pidx_v[bx, pl.ds(j * 16, 16)] = v >> 1
            par_v[bx, pl.ds(j * 16, 16)] = (v & 1) * D
        pltpu.async_copy(tp_hbm.at[pidx_v.at[bx]], rows_v.at[bx],
                         sem_g.at[bx])

    def stage_b(u, bx):
        h = u >> 6
        bc2 = u & (UB - 1)
        pltpu.make_async_copy(tp_hbm.at[pidx_v.at[bx]], rows_v.at[bx],
                              sem_g.at[bx]).wait()
        rows2 = rows_v.at[bx]

        def jbody(jb, carry):
            par = par_v[bx, pl.ds(jb * 16, 16)]
            ri = rowi + jb * 16
            cjb = (jb >> 3) * 1024 + (jb & 7) * 16
            for dg in range(8):
                for ds in range(8):
                    vals = plsc.load_gather(rows2, [ri, par + (dg * 8 + ds)])
                    obuf[bx, pl.ds(dg * 2048 + ds * 128 + cjb, 16)] = vals
            return carry

        lax.fori_loop(0, CHUNK // 16, jbody, 0)
        base = h * HSTRIDE + bc2 * 2048
        for dg in range(8):
            pltpu.async_copy(obuf.at[bx, pl.ds(dg * 2048, 2048)],
                             out_hbm.at[pl.ds(base + dg * 131072, 2048)],
                             sem_o.at[bx])

    def wait_write(u, bx):
        h = u >> 6
        bc2 = u & (UB - 1)
        base = h * HSTRIDE + bc2 * 2048
        for dg in range(8):
            pltpu.make_async_copy(
                obuf.at[bx, pl.ds(dg * 2048, 2048)],
                out_hbm.at[pl.ds(base + dg * 131072, 2048)],
                sem_o.at[bx]).wait()

    stage_a(u0, 0)

    def step(g, carry):
        bx = g & 1

        @pl.when(g < UPW - 1)
        def _():
            stage_a(u0 + g + 1, (g + 1) & 1)

        @pl.when(g >= 2)
        def _():
            wait_write(u0 + g - 2, bx)

        stage_b(u0 + g, bx)
        return carry

    lax.fori_loop(0, UPW, step, 0)
    wait_write(u0 + UPW - 2, (UPW - 2) % 2)
    wait_write(u0 + UPW - 1, (UPW - 1) % 2)


def kernel(x, table):
    b, h = x.shape
    flat = x.T.reshape(h * b)
    tp = table.reshape(table.shape[0] // 2, 2 * D)
    mesh = plsc.VectorSubcoreMesh(core_axis_name="c", subcore_axis_name="s")
    run = pl.kernel(
        _emb_body,
        out_type=jax.ShapeDtypeStruct((h * HSTRIDE,), jnp.float32),
        mesh=mesh,
        scratch_types=[
            pltpu.VMEM((NBUF, CHUNK), jnp.int32),
            pltpu.VMEM((NBUF, CHUNK), jnp.int32),
            pltpu.VMEM((NBUF, CHUNK), jnp.int32),
            pltpu.VMEM((NBUF, CHUNK, 2 * D), jnp.float32),
            pltpu.VMEM((NBUF, OBUF), jnp.float32),
            pltpu.SemaphoreType.DMA((NBUF,)),
            pltpu.SemaphoreType.DMA((NBUF,)),
        ],
        compiler_params=pltpu.CompilerParams(use_tc_tiling_on_sc=False,
                                             needs_layout_passes=False),
    )
    out5 = run(flat, tp).reshape(h, 8, 128, 8, 128)
    return out5.transpose(2, 4, 0, 1, 3).reshape(b, h, D)


# speedup vs baseline: 1.0097x; 1.0097x over previous
import jax
import jax.numpy as jnp
from jax import lax
from jax.experimental import pallas as pl
from jax.experimental.pallas import tpu as pltpu
from jax.experimental.pallas import tpu_sc as plsc

D = 64
NC, NS = 2, 16
NW = NC * NS
BATCH = 16384
HIST = 50
CHUNK = 256
UB = BATCH // CHUNK
NUNITS = HIST * UB
UPW = NUNITS // NW
NBUF = 2
OBUF = 16384
HSTRIDE = 8 * 128 * 8 * 128


def _emb_body(idx_hbm, tp_hbm, out_hbm, idx_v, rows_v, obuf,
              sem_g, sem_o):
    wid = lax.axis_index("s") * NC + lax.axis_index("c")
    u0 = wid * UPW
    rowi = lax.iota(jnp.int32, 16)
    zero = jnp.zeros((16,), jnp.int32)
    dcols = [zero + d for d in range(D)]

    def stage_a(u, bx):
        h = u >> 6
        bc2 = u & (UB - 1)
        i0 = h * BATCH + bc2 * CHUNK
        pltpu.sync_copy(idx_hbm.at[pl.ds(i0, CHUNK)], idx_v.at[bx])
        pltpu.async_copy(tp_hbm.at[idx_v.at[bx]], rows_v.at[bx],
                         sem_g.at[bx])

    def stage_b(u, bx):
        h = u >> 6
        bc2 = u & (UB - 1)
        pltpu.make_async_copy(tp_hbm.at[idx_v.at[bx]], rows_v.at[bx],
                              sem_g.at[bx]).wait()
        rows2 = rows_v.at[bx]

        def jbody(jb, carry):
            ri = rowi + jb * 16
            cjb = (jb >> 3) * 1024 + (jb & 7) * 16
            for dg in range(8):
                for ds in range(8):
                    d = dg * 8 + ds
                    vals = plsc.load_gather(rows2, [ri, dcols[d]])
                    obuf[bx, pl.ds(dg * 2048 + ds * 128 + cjb, 16)] = vals
            return carry

        lax.fori_loop(0, CHUNK // 16, jbody, 0)
        base = h * HSTRIDE + bc2 * 2048
        for dg in range(8):
            pltpu.async_copy(obuf.at[bx, pl.ds(dg * 2048, 2048)],
                             out_hbm.at[pl.ds(base + dg * 131072, 2048)],
                             sem_o.at[bx])

    def wait_write(u, bx):
        h = u >> 6
        bc2 = u & (UB - 1)
        base = h * HSTRIDE + bc2 * 2048
        for dg in range(8):
            pltpu.make_async_copy(
                obuf.at[bx, pl.ds(dg * 2048, 2048)],
                out_hbm.at[pl.ds(base + dg * 131072, 2048)],
                sem_o.at[bx]).wait()

    stage_a(u0, 0)

    def step(g, carry):
        bx = g & 1

        @pl.when(g < UPW - 1)
        def _():
            stage_a(u0 + g + 1, (g + 1) & 1)

        @pl.when(g >= 2)
        def _():
            wait_write(u0 + g - 2, bx)

        stage_b(u0 + g, bx)
        return carry

    lax.fori_loop(0, UPW, step, 0)
    wait_write(u0 + UPW - 2, (UPW - 2) % 2)
    wait_write(u0 + UPW - 1, (UPW - 1) % 2)


def kernel(x, table):
    b, h = x.shape
    flat = x.T.reshape(h * b)
    mesh = plsc.VectorSubcoreMesh(core_axis_name="c", subcore_axis_name="s")
    run = pl.kernel(
        _emb_body,
        out_type=jax.ShapeDtypeStruct((h * HSTRIDE,), jnp.float32),
        mesh=mesh,
        scratch_types=[
            pltpu.VMEM((NBUF, CHUNK), jnp.int32),
            pltpu.VMEM((NBUF, CHUNK, D), jnp.float32),
            pltpu.VMEM((NBUF, OBUF), jnp.float32),
            pltpu.SemaphoreType.DMA((NBUF,)),
            pltpu.SemaphoreType.DMA((NBUF,)),
        ],
        compiler_params=pltpu.CompilerParams(use_tc_tiling_on_sc=False,
                                             needs_layout_passes=False,
                                             disable_bounds_checks=True),
    )
    out5 = run(flat, table).reshape(h, 8, 128, 8, 128)
    return out5.transpose(2, 4, 0, 1, 3).reshape(b, h, D)


# speedup vs baseline: 1.2420x; 1.2300x over previous
import jax
import jax.numpy as jnp
from jax import lax
from jax.experimental import pallas as pl
from jax.experimental.pallas import tpu as pltpu
from jax.experimental.pallas import tpu_sc as plsc

D = 64
NC, NS = 2, 16
NW = NC * NS
BATCH = 16384
HIST = 50
CHUNK = 256
UB = BATCH // CHUNK
NUNITS = HIST * UB
UPW = NUNITS // NW
NBUF = 2
OBUF = 16384
HSTRIDE = 8 * 128 * 8 * 128
PAD = 66


def _emb_body(idx_hbm, tp_hbm, out_hbm, idx_v, rows_v, rpad, obuf,
              sem_g, sem_o):
    wid = lax.axis_index("s") * NC + lax.axis_index("c")
    u0 = wid * UPW
    rowi = lax.iota(jnp.int32, 16)
    zero = jnp.zeros((16,), jnp.int32)
    dcols = [zero + d for d in range(D)]

    def stage_a(u, bx):
        h = u >> 6
        bc2 = u & (UB - 1)
        i0 = h * BATCH + bc2 * CHUNK
        pltpu.sync_copy(idx_hbm.at[pl.ds(i0, CHUNK)], idx_v.at[bx])
        pltpu.async_copy(tp_hbm.at[idx_v.at[bx]], rows_v.at[bx],
                         sem_g.at[bx])

    def stage_b(u, bx):
        h = u >> 6
        bc2 = u & (UB - 1)
        pltpu.make_async_copy(tp_hbm.at[idx_v.at[bx]], rows_v.at[bx],
                              sem_g.at[bx]).wait()

        def rbody(rb, carry):
            for r4 in range(4):
                for q in range(4):
                    rpad[pl.ds((rb * 4 + r4) * PAD + q * 16, 16)] = \
                        rows_v[bx, rb * 4 + r4, pl.ds(q * 16, 16)]
            return carry

        lax.fori_loop(0, CHUNK // 4, rbody, 0)

        def jbody(jb, carry):
            ri66 = (rowi + jb * 16) * PAD
            cjb = (jb >> 3) * 1024 + (jb & 7) * 16
            for dg in range(8):
                for ds in range(8):
                    d = dg * 8 + ds
                    vals = plsc.load_gather(rpad, [ri66 + dcols[d]])
                    obuf[bx, pl.ds(dg * 2048 + ds * 128 + cjb, 16)] = vals
            return carry

        lax.fori_loop(0, CHUNK // 16, jbody, 0)
        base = h * HSTRIDE + bc2 * 2048
        for dg in range(8):
            pltpu.async_copy(obuf.at[bx, pl.ds(dg * 2048, 2048)],
                             out_hbm.at[pl.ds(base + dg * 131072, 2048)],
                             sem_o.at[bx])

    def wait_write(u, bx):
        h = u >> 6
        bc2 = u & (UB - 1)
        base = h * HSTRIDE + bc2 * 2048
        for dg in range(8):
            pltpu.make_async_copy(
                obuf.at[bx, pl.ds(dg * 2048, 2048)],
                out_hbm.at[pl.ds(base + dg * 131072, 2048)],
                sem_o.at[bx]).wait()

    stage_a(u0, 0)

    def step(g, carry):
        bx = g & 1

        @pl.when(g < UPW - 1)
        def _():
            stage_a(u0 + g + 1, (g + 1) & 1)

        @pl.when(g >= 2)
        def _():
            wait_write(u0 + g - 2, bx)

        stage_b(u0 + g, bx)
        return carry

    lax.fori_loop(0, UPW, step, 0)
    wait_write(u0 + UPW - 2, (UPW - 2) % 2)
    wait_write(u0 + UPW - 1, (UPW - 1) % 2)


def kernel(x, table):
    b, h = x.shape
    flat = x.T.reshape(h * b)
    mesh = plsc.VectorSubcoreMesh(core_axis_name="c", subcore_axis_name="s")
    run = pl.kernel(
        _emb_body,
        out_type=jax.ShapeDtypeStruct((h * HSTRIDE,), jnp.float32),
        mesh=mesh,
        scratch_types=[
            pltpu.VMEM((NBUF, CHUNK), jnp.int32),
            pltpu.VMEM((NBUF, CHUNK, D), jnp.float32),
            pltpu.VMEM((CHUNK * PAD,), jnp.float32),
            pltpu.VMEM((NBUF, OBUF), jnp.float32),
            pltpu.SemaphoreType.DMA((NBUF,)),
            pltpu.SemaphoreType.DMA((NBUF,)),
        ],
        compiler_params=pltpu.CompilerParams(use_tc_tiling_on_sc=False,
                                             needs_layout_passes=False,
                                             disable_bounds_checks=True),
    )
    out5 = run(flat, table).reshape(h, 8, 128, 8, 128)
    return out5.transpose(2, 4, 0, 1, 3).reshape(b, h, D)


# speedup vs baseline: 2.0708x; 1.6674x over previous
import jax
import jax.numpy as jnp
from jax import lax
from jax.experimental import pallas as pl
from jax.experimental.pallas import tpu as pltpu
from jax.experimental.pallas import tpu_sc as plsc

D = 64
NC, NS = 2, 16
NW = NC * NS
BATCH = 16384
HIST = 50
CHUNK = 256
UB = BATCH // CHUNK
NUNITS = HIST * UB
UPW = NUNITS // NW
NBUF = 2
OBUF = 16384
HSTRIDE = 8 * 128 * 8 * 128
PAD = 66


def _emb_body(idx_hbm, tp_hbm, out_hbm, idx_v, rows_v, rpad, obuf,
              sem_g, sem_o):
    wid = lax.axis_index("s") * NC + lax.axis_index("c")
    u0 = wid * UPW
    rowi = lax.iota(jnp.int32, 16)
    zero = jnp.zeros((16,), jnp.int32)
    dcols = [zero + d for d in range(D)]

    def stage_a(u, bx):
        h = u >> 6
        bc2 = u & (UB - 1)
        i0 = h * BATCH + bc2 * CHUNK
        pltpu.sync_copy(idx_hbm.at[pl.ds(i0, CHUNK)], idx_v.at[bx])
        pltpu.async_copy(tp_hbm.at[idx_v.at[bx]], rows_v.at[bx],
                         sem_g.at[bx])

    def stage_b(u, bx):
        h = u >> 6
        bc2 = u & (UB - 1)
        pltpu.make_async_copy(tp_hbm.at[idx_v.at[bx]], rows_v.at[bx],
                              sem_g.at[bx]).wait()

        def rbody(rb, carry):
            vals = [rows_v[bx, rb * 4 + r4, pl.ds(q * 16, 16)]
                    for r4 in range(4) for q in range(4)]
            for r4 in range(4):
                for q in range(4):
                    rpad[pl.ds((rb * 4 + r4) * PAD + q * 16, 16)] = \
                        vals[r4 * 4 + q]
            return carry

        lax.fori_loop(0, CHUNK // 4, rbody, 0)

        def jbody(jb, carry):
            ri66 = (rowi + jb * 16) * PAD
            cjb = (jb >> 3) * 1024 + (jb & 7) * 16
            for dg in range(8):
                vals = [plsc.load_gather(rpad, [ri66 + dcols[dg * 8 + ds]])
                        for ds in range(8)]
                for ds in range(8):
                    obuf[bx, pl.ds(dg * 2048 + ds * 128 + cjb, 16)] = vals[ds]
            return carry

        lax.fori_loop(0, CHUNK // 16, jbody, 0)
        base = h * HSTRIDE + bc2 * 2048
        for dg in range(8):
            pltpu.async_copy(obuf.at[bx, pl.ds(dg * 2048, 2048)],
                             out_hbm.at[pl.ds(base + dg * 131072, 2048)],
                             sem_o.at[bx])

    def wait_write(u, bx):
        h = u >> 6
        bc2 = u & (UB - 1)
        base = h * HSTRIDE + bc2 * 2048
        for dg in range(8):
            pltpu.make_async_copy(
                obuf.at[bx, pl.ds(dg * 2048, 2048)],
                out_hbm.at[pl.ds(base + dg * 131072, 2048)],
                sem_o.at[bx]).wait()

    stage_a(u0, 0)

    def step(g, carry):
        bx = g & 1

        @pl.when(g < UPW - 1)
        def _():
            stage_a(u0 + g + 1, (g + 1) & 1)

        @pl.when(g >= 2)
        def _():
            wait_write(u0 + g - 2, bx)

        stage_b(u0 + g, bx)
        return carry

    lax.fori_loop(0, UPW, step, 0)
    wait_write(u0 + UPW - 2, (UPW - 2) % 2)
    wait_write(u0 + UPW - 1, (UPW - 1) % 2)


def kernel(x, table):
    b, h = x.shape
    flat = x.T.reshape(h * b)
    mesh = plsc.VectorSubcoreMesh(core_axis_name="c", subcore_axis_name="s")
    run = pl.kernel(
        _emb_body,
        out_type=jax.ShapeDtypeStruct((h * HSTRIDE,), jnp.float32),
        mesh=mesh,
        scratch_types=[
            pltpu.VMEM((NBUF, CHUNK), jnp.int32),
            pltpu.VMEM((NBUF, CHUNK, D), jnp.float32),
            pltpu.VMEM((CHUNK * PAD,), jnp.float32),
            pltpu.VMEM((NBUF, OBUF), jnp.float32),
            pltpu.SemaphoreType.DMA((NBUF,)),
            pltpu.SemaphoreType.DMA((NBUF,)),
        ],
        compiler_params=pltpu.CompilerParams(use_tc_tiling_on_sc=False,
                                             needs_layout_passes=False,
                                             disable_bounds_checks=True),
    )
    out5 = run(flat, table).reshape(h, 8, 128, 8, 128)
    return out5.transpose(2, 4, 0, 1, 3).reshape(b, h, D)


# speedup vs baseline: 2.7423x; 1.3243x over previous
import jax
import jax.numpy as jnp
from jax import lax
from jax.experimental import pallas as pl
from jax.experimental.pallas import tpu as pltpu
from jax.experimental.pallas import tpu_sc as plsc

D = 64
NC, NS = 2, 16
NW = NC * NS
BATCH = 16384
HIST = 50
CHUNK = 256
UB = BATCH // CHUNK
NUNITS = HIST * UB
UPW = NUNITS // NW
NBUF = 2
OBUF = 16384
HSTRIDE = 8 * 128 * 8 * 128
PAD = 66


NVB = 1000000 // 128
VB_PW = NVB // NW
VB_REM = NVB - VB_PW * NW
VTAIL = 1000000 - NVB * 128
IPAD = 130


def _detile_body(tt_hbm, tp_hbm, inblk, ipad, obuf, tail_v, sem_i, sem_o):
    wid = lax.axis_index("s") * NC + lax.axis_index("c")
    rowi = lax.iota(jnp.int32, 16)
    pats = [(q * 16 + rowi) * IPAD for q in range(4)]

    def in_copy(c0, bx):
        return pltpu.make_async_copy(tt_hbm.at[:, pl.ds(c0, 128)],
                                     inblk.at[bx], sem_i.at[bx])

    def out_copy(c0, bx):
        return pltpu.make_async_copy(obuf.at[bx],
                                     tp_hbm.at[pl.ds(c0 * D, 8192)],
                                     sem_o.at[bx])

    def process_block(bx, nv):
        def dbody(db, carry):
            vals = []
            for d4 in range(4):
                for q in range(nv // 16):
                    vals.append(inblk[bx, db * 4 + d4, pl.ds(q * 16, 16)])
            i = 0
            for d4 in range(4):
                for q in range(nv // 16):
                    ipad[pl.ds((db * 4 + d4) * IPAD + q * 16, 16)] = vals[i]
                    i += 1
            return carry
        lax.fori_loop(0, D // 4, dbody, 0)

        def vbody(vb, carry):
            vals = []
            for v4 in range(4):
                v = vb * 4 + v4
                for q in range(4):
                    vals.append(plsc.load_gather(ipad, [pats[q] + v]))
            i = 0
            for v4 in range(4):
                v = vb * 4 + v4
                for q in range(4):
                    obuf[bx, pl.ds(v * D + q * 16, 16)] = vals[i]
                    i += 1
            return carry
        lax.fori_loop(0, nv // 4, vbody, 0)

    c_base = wid * VB_PW * 128
    in_copy(c_base, 0).start()

    def step(k, carry):
        c0 = c_base + k * 128
        bx = k & 1

        @pl.when(k < VB_PW - 1)
        def _():
            in_copy(c0 + 128, (k + 1) & 1).start()

        @pl.when(k >= 2)
        def _():
            out_copy(c0 - 256, bx).wait()

        in_copy(c0, bx).wait()
        process_block(bx, 128)
        out_copy(c0, bx).start()
        return carry

    lax.fori_loop(0, VB_PW, step, 0)
    out_copy(c_base + (VB_PW - 2) * 128, (VB_PW - 2) & 1).wait()
    out_copy(c_base + (VB_PW - 1) * 128, (VB_PW - 1) & 1).wait()

    for widx in range(VB_REM):
        @pl.when(wid == widx)
        def _():
            c0 = (NVB - VB_REM + widx) * 128
            in_copy(c0, 0).start()
            in_copy(c0, 0).wait()
            process_block(0, 128)
            out_copy(c0, 0).start()
            out_copy(c0, 0).wait()

    @pl.when(wid == VB_REM)
    def _():
        c0 = NVB * 128
        src = tt_hbm.at[:, pl.ds(c0, VTAIL)]
        pltpu.make_async_copy(src, tail_v, sem_i.at[0]).start()
        pltpu.make_async_copy(src, tail_v, sem_i.at[0]).wait()

        def dbody(db, carry):
            vals = []
            for d4 in range(4):
                for q in range(VTAIL // 16):
                    vals.append(tail_v[db * 4 + d4, pl.ds(q * 16, 16)])
            i = 0
            for d4 in range(4):
                for q in range(VTAIL // 16):
                    ipad[pl.ds((db * 4 + d4) * IPAD + q * 16, 16)] = vals[i]
                    i += 1
            return carry
        lax.fori_loop(0, D // 4, dbody, 0)

        def vbody(vb, carry):
            vals = []
            for v4 in range(4):
                v = vb * 4 + v4
                for q in range(4):
                    vals.append(plsc.load_gather(ipad, [pats[q] + v]))
            i = 0
            for v4 in range(4):
                v = vb * 4 + v4
                for q in range(4):
                    obuf[0, pl.ds(v * D + q * 16, 16)] = vals[i]
                    i += 1
            return carry
        lax.fori_loop(0, VTAIL // 4, vbody, 0)
        osrc = obuf.at[0, pl.ds(0, VTAIL * D)]
        odst = tp_hbm.at[pl.ds(c0 * D, VTAIL * D)]
        pltpu.make_async_copy(osrc, odst, sem_o.at[0]).start()
        pltpu.make_async_copy(osrc, odst, sem_o.at[0]).wait()


def _emb_body(idx_hbm, tp_hbm, out_hbm, idx_v, rows_v, rpad, obuf,
              sem_g, sem_o):
    wid = lax.axis_index("s") * NC + lax.axis_index("c")
    u0 = wid * UPW
    rowi = lax.iota(jnp.int32, 16)
    zero = jnp.zeros((16,), jnp.int32)
    dcols = [zero + d for d in range(D)]

    def stage_a(u, bx):
        h = u >> 6
        bc2 = u & (UB - 1)
        i0 = h * BATCH + bc2 * CHUNK
        pltpu.sync_copy(idx_hbm.at[pl.ds(i0, CHUNK)], idx_v.at[bx])
        pltpu.async_copy(tp_hbm.at[idx_v.at[bx]], rows_v.at[bx],
                         sem_g.at[bx])

    def stage_b(u, bx):
        h = u >> 6
        bc2 = u & (UB - 1)
        pltpu.make_async_copy(tp_hbm.at[idx_v.at[bx]], rows_v.at[bx],
                              sem_g.at[bx]).wait()

        def rbody(rb, carry):
            vals = [rows_v[bx, rb * 4 + r4, pl.ds(q * 16, 16)]
                    for r4 in range(4) for q in range(4)]
            for r4 in range(4):
                for q in range(4):
                    rpad[pl.ds((rb * 4 + r4) * PAD + q * 16, 16)] = \
                        vals[r4 * 4 + q]
            return carry

        lax.fori_loop(0, CHUNK // 4, rbody, 0)

        def jbody(jb, carry):
            ri66 = (rowi + jb * 16) * PAD
            cjb = (jb >> 3) * 1024 + (jb & 7) * 16
            for dg in range(8):
                vals = [plsc.load_gather(rpad, [ri66 + dcols[dg * 8 + ds]])
                        for ds in range(8)]
                for ds in range(8):
                    obuf[bx, pl.ds(dg * 2048 + ds * 128 + cjb, 16)] = vals[ds]
            return carry

        lax.fori_loop(0, CHUNK // 16, jbody, 0)
        base = h * HSTRIDE + bc2 * 2048
        for dg in range(8):
            pltpu.async_copy(obuf.at[bx, pl.ds(dg * 2048, 2048)],
                             out_hbm.at[pl.ds(base + dg * 131072, 2048)],
                             sem_o.at[bx])

    def wait_write(u, bx):
        h = u >> 6
        bc2 = u & (UB - 1)
        base = h * HSTRIDE + bc2 * 2048
        for dg in range(8):
            pltpu.make_async_copy(
                obuf.at[bx, pl.ds(dg * 2048, 2048)],
                out_hbm.at[pl.ds(base + dg * 131072, 2048)],
                sem_o.at[bx]).wait()

    stage_a(u0, 0)

    def step(g, carry):
        bx = g & 1

        @pl.when(g < UPW - 1)
        def _():
            stage_a(u0 + g + 1, (g + 1) & 1)

        @pl.when(g >= 2)
        def _():
            wait_write(u0 + g - 2, bx)

        stage_b(u0 + g, bx)
        return carry

    lax.fori_loop(0, UPW, step, 0)
    wait_write(u0 + UPW - 2, (UPW - 2) % 2)
    wait_write(u0 + UPW - 1, (UPW - 1) % 2)


def kernel(x, table):
    b, h = x.shape
    flat = x.T.reshape(h * b)
    mesh = plsc.VectorSubcoreMesh(core_axis_name="c", subcore_axis_name="s")

    detile = pl.kernel(
        _detile_body,
        out_type=jax.ShapeDtypeStruct((1000000 * D,), jnp.float32),
        mesh=mesh,
        scratch_types=[
            pltpu.VMEM((NBUF, D, 128), jnp.float32),
            pltpu.VMEM((D * IPAD,), jnp.float32),
            pltpu.VMEM((NBUF, 8192), jnp.float32),
            pltpu.VMEM((D, VTAIL), jnp.float32),
            pltpu.SemaphoreType.DMA((NBUF,)),
            pltpu.SemaphoreType.DMA((NBUF,)),
        ],
        compiler_params=pltpu.CompilerParams(use_tc_tiling_on_sc=True,
                                             needs_layout_passes=False,
                                             disable_bounds_checks=True),
    )
    tp = detile(table.T).reshape(1000000, D)

    run = pl.kernel(
        _emb_body,
        out_type=jax.ShapeDtypeStruct((h * HSTRIDE,), jnp.float32),
        mesh=mesh,
        scratch_types=[
            pltpu.VMEM((NBUF, CHUNK), jnp.int32),
            pltpu.VMEM((NBUF, CHUNK, D), jnp.float32),
            pltpu.VMEM((CHUNK * PAD,), jnp.float32),
            pltpu.VMEM((NBUF, OBUF), jnp.float32),
            pltpu.SemaphoreType.DMA((NBUF,)),
            pltpu.SemaphoreType.DMA((NBUF,)),
        ],
        compiler_params=pltpu.CompilerParams(use_tc_tiling_on_sc=False,
                                             needs_layout_passes=False,
                                             disable_bounds_checks=True),
    )
    out5 = run(flat, tp).reshape(h, 8, 128, 8, 128)
    return out5.transpose(2, 4, 0, 1, 3).reshape(b, h, D)
